# baseline (speedup 1.0000x reference)
import jax
import jax.numpy as jnp
from jax import lax
from jax.experimental import pallas as pl
from jax.experimental.pallas import tpu as pltpu
from jax.experimental.pallas import tpu_sc as plsc

B = 16384
L = 20
E = 64
V = 1000000
NC = 2
NS = 16
NW = NC * NS
BPW = B // NW
C = 16
STEPS = BPW // C
PK = 768


def _compute_chunk(s, pack_v, h_v, u_v, ob_v, liota, rowbase):
    tcol = pack_v[pl.ds(16, 16)]
    ccol = [plsc.load_gather(pack_v, [352 + rowbase[l]]) for l in range(L)]

    def estep(e, accs):
        eidx = (liota + e) & (E - 1)
        he = plsc.load_gather(h_v, [liota, tcol + eidx])
        return tuple(
            acc + he * plsc.load_gather(u_v, [rowbase[l], ccol[l] + eidx])
            for l, acc in enumerate(accs)
        )

    accs = lax.fori_loop(
        0, E, estep,
        tuple(jnp.zeros((16,), jnp.float32) for _ in range(L)),
        unroll=4)

    obase = s * (C * L)
    for l in range(L):
        sig = 1.0 / (1.0 + jnp.exp(-accs[l]))
        plsc.store_scatter(ob_v, [obase + rowbase[l]], sig)


def _body(pack_hbm, temb_hbm, cemb_hbm, out_hbm,
          pack_v0, pack_v1, h_v0, h_v1, u_v0, u_v1, ob_v,
          sem_p0, sem_p1, sem_h0, sem_h1, sem_u0, sem_u1):
    wid = lax.axis_index("s") * NC + lax.axis_index("c")
    liota = lax.iota(jnp.int32, 16)
    rowbase = [liota * L + l for l in range(L)]
    pbase = wid * (STEPS + 2) * PK

    def pk_slice(s):
        return pack_hbm.at[pl.ds(pbase + s * PK, PK)]

    bufs = ((pack_v0, h_v0, u_v0, sem_p0, sem_h0, sem_u0),
            (pack_v1, h_v1, u_v1, sem_p1, sem_h1, sem_u1))

    def issue_gathers(s_unused, buf):
        pack_v, h_v, u_v, _, sem_h, sem_u = buf
        pltpu.async_copy(temb_hbm.at[pack_v.at[pl.ds(0, 16)]], h_v, sem_h)
        pltpu.async_copy(cemb_hbm.at[pack_v.at[pl.ds(32, 320)]], u_v, sem_u)

    def wait_gathers(buf):
        pack_v, h_v, u_v, _, sem_h, sem_u = buf
        pltpu.make_async_copy(
            temb_hbm.at[pack_v.at[pl.ds(0, 16)]], h_v, sem_h).wait()
        pltpu.make_async_copy(
            cemb_hbm.at[pack_v.at[pl.ds(32, 320)]], u_v, sem_u).wait()

    pltpu.async_copy(pk_slice(0), pack_v0, sem_p0).wait()
    issue_gathers(0, bufs[0])
    pltpu.async_copy(pk_slice(1), pack_v1, sem_p1)

    def phase(s, cur, nxt):
        pack_c, h_c, u_c, _, _, _ = cur
        pack_n, _, _, sem_pn, _, _ = nxt
        _, _, _, sem_pc, _, _ = cur
        wait_gathers(cur)
        pltpu.make_async_copy(pk_slice(s + 1), pack_n, sem_pn).wait()
        issue_gathers(s + 1, nxt)
        pltpu.async_copy(pk_slice(s + 2), pack_c, sem_pc)
        _compute_chunk(s, pack_c, h_c, u_c, ob_v, liota, rowbase)

    def pair(i, _):
        s = 2 * i
        phase(s, bufs[0], bufs[1])
        phase(s + 1, bufs[1], bufs[0])
        return ()

    lax.fori_loop(0, STEPS // 2, pair, ())

    wait_gathers(bufs[0])
    pltpu.make_async_copy(pk_slice(STEPS + 1), pack_v1, sem_p1).wait()

    pltpu.sync_copy(ob_v, out_hbm.at[pl.ds(wid * (BPW * L), BPW * L)])


@jax.jit
def _run(pack, temb2, cemb2):
    mesh = plsc.VectorSubcoreMesh(
        core_axis_name="c", subcore_axis_name="s",
        num_cores=NC, num_subcores=NS)
    f = pl.kernel(
        _body,
        out_type=jax.ShapeDtypeStruct((B * L,), jnp.float32),
        mesh=mesh,
        scratch_types=[
            pltpu.VMEM((PK,), jnp.int32),
            pltpu.VMEM((PK,), jnp.int32),
            pltpu.VMEM((C, 2 * E), jnp.float32),
            pltpu.VMEM((C, 2 * E), jnp.float32),
            pltpu.VMEM((C * L, 2 * E), jnp.float32),
            pltpu.VMEM((C * L, 2 * E), jnp.float32),
            pltpu.VMEM((BPW * L,), jnp.float32),
            pltpu.SemaphoreType.DMA,
            pltpu.SemaphoreType.DMA,
            pltpu.SemaphoreType.DMA,
            pltpu.SemaphoreType.DMA,
            pltpu.SemaphoreType.DMA,
            pltpu.SemaphoreType.DMA,
        ],
        compiler_params=pltpu.CompilerParams(
            needs_layout_passes=False, use_tc_tiling_on_sc=True),
    )
    return f(pack, temb2, cemb2)


def kernel(target_word_id, context_word_ids, target_embeddings,
           context_embeddings):
    tid = target_word_id.reshape(-1).astype(jnp.int32)
    cid = context_word_ids.reshape(-1).astype(jnp.int32)
    th = (tid >> 1).reshape(NW, STEPS, C)
    tc = ((tid & 1) * E).reshape(NW, STEPS, C)
    ch = (cid >> 1).reshape(NW, STEPS, C * L)
    cc = ((cid & 1) * E).reshape(NW, STEPS, C * L)
    zpad = jnp.zeros((NW, STEPS, PK - 2 * C - 2 * C * L), jnp.int32)
    pack = jnp.concatenate([th, tc, ch, cc, zpad], axis=2)
    pack = jnp.concatenate(
        [pack, jnp.zeros((NW, 2, PK), jnp.int32)], axis=1).reshape(-1)
    temb2 = target_embeddings.reshape(V // 2, 2 * E)
    cemb2 = context_embeddings.reshape(V // 2, 2 * E)
    out = _run(pack, temb2, cemb2)
    return out.reshape(B, L)


# speedup vs baseline: 1.2792x; 1.2792x over previous
import jax
import jax.numpy as jnp
from jax import lax
from jax.experimental import pallas as pl
from jax.experimental.pallas import tpu as pltpu
from jax.experimental.pallas import tpu_sc as plsc

B = 16384
L = 20
E = 64
V = 1000000
NC = 2
NS = 16
NW = NC * NS
BPW = B // NW
C = 16
STEPS = BPW // C
PK = 512


def _compute_chunk(s, h_v, u_v, ob_v, liota, rowbase):
    def estep(e, accs):
        eidx = (liota + e) & (E - 1)
        he = plsc.load_gather(h_v, [liota, eidx])
        return tuple(
            acc + he * plsc.load_gather(u_v, [rowbase[l], eidx])
            for l, acc in enumerate(accs)
        )

    accs = lax.fori_loop(
        0, E, estep,
        tuple(jnp.zeros((16,), jnp.float32) for _ in range(L)),
        unroll=4)

    obase = s * (C * L)
    for l in range(L):
        sig = 1.0 / (1.0 + jnp.exp(-accs[l]))
        plsc.store_scatter(ob_v, [obase + rowbase[l]], sig)


def _body(pack_hbm, temb_hbm, cemb_hbm, out_hbm,
          pack_v0, pack_v1, h_v0, h_v1, u_v0, u_v1, ob_v,
          sem_p0, sem_p1, sem_h0, sem_h1, sem_u0, sem_u1):
    wid = lax.axis_index("s") * NC + lax.axis_index("c")
    liota = lax.iota(jnp.int32, 16)
    rowbase = [liota * L + l for l in range(L)]
    pbase = wid * (STEPS + 2) * PK

    def pk_slice(s):
        return pack_hbm.at[pl.ds(pbase + s * PK, PK)]

    bufs = ((pack_v0, h_v0, u_v0, sem_p0, sem_h0, sem_u0),
            (pack_v1, h_v1, u_v1, sem_p1, sem_h1, sem_u1))

    def issue_gathers(s_unused, buf):
        pack_v, h_v, u_v, _, sem_h, sem_u = buf

        hv16 = pack_v[pl.ds(0, 16)]
        for j in range(16):
            pltpu.async_copy(
                temb_hbm.at[pl.ds(hv16[j], 1)], h_v.at[pl.ds(j, 1)], sem_h)

        def issue_u(g, _):
            uv16 = pack_v[pl.ds(32 + g * 16, 16)]
            base = g * 16
            for j in range(16):
                pltpu.async_copy(
                    cemb_hbm.at[pl.ds(uv16[j], 1)],
                    u_v.at[pl.ds(base + j, 1)], sem_u)
            return ()

        lax.fori_loop(0, 20, issue_u, ())

    def wait_gathers(buf):
        pack_v, h_v, u_v, _, sem_h, sem_u = buf
        hv16 = pack_v[pl.ds(0, 16)]
        for j in range(16):
            pltpu.make_async_copy(
                temb_hbm.at[pl.ds(hv16[j], 1)], h_v.at[pl.ds(j, 1)],
                sem_h).wait()

        def wait_u(g, _):
            uv16 = pack_v[pl.ds(32 + g * 16, 16)]
            base = g * 16
            for j in range(16):
                pltpu.make_async_copy(
                    cemb_hbm.at[pl.ds(uv16[j], 1)],
                    u_v.at[pl.ds(base + j, 1)], sem_u).wait()
            return ()

        lax.fori_loop(0, 20, wait_u, ())

    pltpu.async_copy(pk_slice(0), pack_v0, sem_p0).wait()
    issue_gathers(0, bufs[0])
    pltpu.async_copy(pk_slice(1), pack_v1, sem_p1)

    def phase(s, cur, nxt):
        pack_c, h_c, u_c, _, _, _ = cur
        pack_n, _, _, sem_pn, _, _ = nxt
        _, _, _, sem_pc, _, _ = cur
        wait_gathers(cur)
        pltpu.make_async_copy(pk_slice(s + 1), pack_n, sem_pn).wait()
        issue_gathers(s + 1, nxt)
        pltpu.async_copy(pk_slice(s + 2), pack_c, sem_pc)
        _compute_chunk(s, h_c, u_c, ob_v, liota, rowbase)

    def pair(i, _):
        s = 2 * i
        phase(s, bufs[0], bufs[1])
        phase(s + 1, bufs[1], bufs[0])
        return ()

    lax.fori_loop(0, STEPS // 2, pair, ())

    wait_gathers(bufs[0])
    pltpu.make_async_copy(pk_slice(STEPS + 1), pack_v1, sem_p1).wait()

    pltpu.sync_copy(ob_v, out_hbm.at[pl.ds(wid * (BPW * L), BPW * L)])


@jax.jit
def _run(pack, temb, cemb):
    mesh = plsc.VectorSubcoreMesh(
        core_axis_name="c", subcore_axis_name="s",
        num_cores=NC, num_subcores=NS)
    f = pl.kernel(
        _body,
        out_type=jax.ShapeDtypeStruct((B * L,), jnp.float32),
        mesh=mesh,
        scratch_types=[
            pltpu.VMEM((PK,), jnp.int32),
            pltpu.VMEM((PK,), jnp.int32),
            pltpu.VMEM((C, E), jnp.float32),
            pltpu.VMEM((C, E), jnp.float32),
            pltpu.VMEM((C * L, E), jnp.float32),
            pltpu.VMEM((C * L, E), jnp.float32),
            pltpu.VMEM((BPW * L,), jnp.float32),
            pltpu.SemaphoreType.DMA,
            pltpu.SemaphoreType.DMA,
            pltpu.SemaphoreType.DMA,
            pltpu.SemaphoreType.DMA,
            pltpu.SemaphoreType.DMA,
            pltpu.SemaphoreType.DMA,
        ],
        compiler_params=pltpu.CompilerParams(
            needs_layout_passes=False, use_tc_tiling_on_sc=True),
    )
    return f(pack, temb, cemb)


def kernel(target_word_id, context_word_ids, target_embeddings,
           context_embeddings):
    tid = target_word_id.reshape(-1).astype(jnp.int32)
    cid = context_word_ids.reshape(-1).astype(jnp.int32)
    th = tid.reshape(NW, STEPS, C)
    ch = cid.reshape(NW, STEPS, C * L)
    zpad = jnp.zeros((NW, STEPS, PK - C - 16 - C * L), jnp.int32)
    zpad16 = jnp.zeros((NW, STEPS, 16), jnp.int32)
    pack = jnp.concatenate([th, zpad16, ch, zpad], axis=2)
    pack = jnp.concatenate(
        [pack, jnp.zeros((NW, 2, PK), jnp.int32)], axis=1).reshape(-1)
    out = _run(pack, target_embeddings, context_embeddings)
    return out.reshape(B, L)


# speedup vs baseline: 1.2946x; 1.0121x over previous
import jax
import jax.numpy as jnp
from jax import lax
from jax.experimental import pallas as pl
from jax.experimental.pallas import tpu as pltpu
from jax.experimental.pallas import tpu_sc as plsc

B = 16384
L = 20
E = 64
V = 1000000
NC = 2
NS = 16
NW = NC * NS
BPW = B // NW
C = 16
STEPS = BPW // C
PK = 512


def _compute_chunk(s, h_v, u_v, ob_v, liota, rowbase):
    def estep(e, accs):
        eidx = (liota + e) & (E - 1)
        he = plsc.load_gather(h_v, [liota, eidx])
        return tuple(
            acc + he * plsc.load_gather(u_v, [rowbase[l], eidx])
            for l, acc in enumerate(accs)
        )

    accs = lax.fori_loop(
        0, E, estep,
        tuple(jnp.zeros((16,), jnp.float32) for _ in range(L)),
        unroll=4)

    obase = s * (C * L)
    for l in range(L):
        sig = 1.0 / (1.0 + jnp.exp(-accs[l]))
        plsc.store_scatter(ob_v, [obase + rowbase[l]], sig)


def _body(pack_hbm, temb_hbm, cemb_hbm, out_hbm,
          pack_v0, pack_v1, h_v0, h_v1, u_v0, u_v1, ob_v,
          sem_p0, sem_p1, sem_h0, sem_h1, sem_u0, sem_u1):
    wid = lax.axis_index("s") * NC + lax.axis_index("c")
    liota = lax.iota(jnp.int32, 16)
    rowbase = [liota * L + l for l in range(L)]
    pbase = wid * (STEPS + 2) * PK

    def pk_slice(s):
        return pack_hbm.at[pl.ds(pbase + s * PK, PK)]

    bufs = ((pack_v0, h_v0, u_v0, sem_p0, sem_h0, sem_u0),
            (pack_v1, h_v1, u_v1, sem_p1, sem_h1, sem_u1))

    def issue_gathers(s_unused, buf):
        pack_v, h_v, u_v, _, sem_h, sem_u = buf

        hv16 = pack_v[pl.ds(0, 16)]
        for j in range(16):
            pltpu.async_copy(
                temb_hbm.at[pl.ds(hv16[j], 1)], h_v.at[pl.ds(j, 1)], sem_h)

        def issue_u(g, _):
            uv16 = pack_v[pl.ds(32 + g * 16, 16)]
            base = g * 16
            for j in range(16):
                pltpu.async_copy(
                    cemb_hbm.at[pl.ds(uv16[j], 1)],
                    u_v.at[pl.ds(base + j, 1)], sem_u)
            return ()

        lax.fori_loop(0, 20, issue_u, ())

    def wait_gathers(buf):
        pack_v, h_v, u_v, _, sem_h, sem_u = buf
        pltpu.make_async_copy(temb_hbm.at[pl.ds(0, 16)], h_v, sem_h).wait()
        pltpu.make_async_copy(cemb_hbm.at[pl.ds(0, 320)], u_v, sem_u).wait()

    pltpu.async_copy(pk_slice(0), pack_v0, sem_p0).wait()
    issue_gathers(0, bufs[0])
    pltpu.async_copy(pk_slice(1), pack_v1, sem_p1)

    def phase(s, cur, nxt):
        pack_c, h_c, u_c, _, _, _ = cur
        pack_n, _, _, sem_pn, _, _ = nxt
        _, _, _, sem_pc, _, _ = cur
        wait_gathers(cur)
        pltpu.make_async_copy(pk_slice(s + 1), pack_n, sem_pn).wait()
        issue_gathers(s + 1, nxt)
        pltpu.async_copy(pk_slice(s + 2), pack_c, sem_pc)
        _compute_chunk(s, h_c, u_c, ob_v, liota, rowbase)

    def pair(i, _):
        s = 2 * i
        phase(s, bufs[0], bufs[1])
        phase(s + 1, bufs[1], bufs[0])
        return ()

    lax.fori_loop(0, STEPS // 2, pair, ())

    wait_gathers(bufs[0])
    pltpu.make_async_copy(pk_slice(STEPS + 1), pack_v1, sem_p1).wait()

    pltpu.sync_copy(ob_v, out_hbm.at[pl.ds(wid * (BPW * L), BPW * L)])


@jax.jit
def _run(pack, temb, cemb):
    mesh = plsc.VectorSubcoreMesh(
        core_axis_name="c", subcore_axis_name="s",
        num_cores=NC, num_subcores=NS)
    f = pl.kernel(
        _body,
        out_type=jax.ShapeDtypeStruct((B * L,), jnp.float32),
        mesh=mesh,
        scratch_types=[
            pltpu.VMEM((PK,), jnp.int32),
            pltpu.VMEM((PK,), jnp.int32),
            pltpu.VMEM((C, E), jnp.float32),
            pltpu.VMEM((C, E), jnp.float32),
            pltpu.VMEM((C * L, E), jnp.float32),
            pltpu.VMEM((C * L, E), jnp.float32),
            pltpu.VMEM((BPW * L,), jnp.float32),
            pltpu.SemaphoreType.DMA,
            pltpu.SemaphoreType.DMA,
            pltpu.SemaphoreType.DMA,
            pltpu.SemaphoreType.DMA,
            pltpu.SemaphoreType.DMA,
            pltpu.SemaphoreType.DMA,
        ],
        compiler_params=pltpu.CompilerParams(
            needs_layout_passes=False, use_tc_tiling_on_sc=True),
    )
    return f(pack, temb, cemb)


def kernel(target_word_id, context_word_ids, target_embeddings,
           context_embeddings):
    tid = target_word_id.reshape(-1).astype(jnp.int32)
    cid = context_word_ids.reshape(-1).astype(jnp.int32)
    th = tid.reshape(NW, STEPS, C)
    ch = cid.reshape(NW, STEPS, C * L)
    zpad = jnp.zeros((NW, STEPS, PK - C - 16 - C * L), jnp.int32)
    zpad16 = jnp.zeros((NW, STEPS, 16), jnp.int32)
    pack = jnp.concatenate([th, zpad16, ch, zpad], axis=2)
    pack = jnp.concatenate(
        [pack, jnp.zeros((NW, 2, PK), jnp.int32)], axis=1).reshape(-1)
    out = _run(pack, target_embeddings, context_embeddings)
    return out.reshape(B, L)
